# baseline (speedup 1.0000x reference)
import functools

import jax
import jax.numpy as jnp
from jax import lax
from jax.experimental import pallas as pl
from jax.experimental.pallas import tpu as pltpu
from jax.experimental.pallas import tpu_sc as plsc

_VOCAB = 1000000
_DIM = 32
_BATCH = 16384
_HIST = 50
_NC = 2
_NS = 16
_NW = _NC * _NS
_BW = _BATCH // _NW
_L = 16

_mesh = plsc.VectorSubcoreMesh(core_axis_name="c", subcore_axis_name="s")


@functools.partial(
    pl.kernel,
    out_type=jax.ShapeDtypeStruct((_HIST, _DIM, _BATCH), jnp.float32),
    mesh=_mesh,
    scratch_types=[
        pltpu.VMEM((_HIST, _BW), jnp.int32),
        pltpu.VMEM((2, _BW, _DIM), jnp.float32),
        pltpu.VMEM((2, _DIM, _BW), jnp.float32),
        pltpu.SemaphoreType.DMA,
        pltpu.SemaphoreType.DMA,
        pltpu.SemaphoreType.DMA,
        pltpu.SemaphoreType.DMA,
    ],
    compiler_params=pltpu.CompilerParams(
        use_tc_tiling_on_sc=False, needs_layout_passes=False
    ),
)
def _gather_kernel(ids_hbm, table_hbm, out_hbm, idx_v, rows_v, cols_v,
                   g0, g1, s0, s1):
    gsems = (g0, g1)
    ssems = (s0, s1)
    wid = lax.axis_index("s") * _NC + lax.axis_index("c")
    b0 = wid * _BW
    pltpu.sync_copy(ids_hbm.at[:, pl.ds(b0, _BW)], idx_v)

    def fire_gather(h, b):
        pltpu.async_copy(table_hbm.at[idx_v.at[h]], rows_v.at[b], gsems[b])

    def wait_gather(b):
        pltpu.make_async_copy(
            table_hbm.at[idx_v.at[0]], rows_v.at[b], gsems[b]
        ).wait()

    def fire_store(h, b):
        pltpu.async_copy(cols_v.at[b], out_hbm.at[h, :, pl.ds(b0, _BW)],
                         ssems[b])

    def wait_store(b):
        pltpu.make_async_copy(
            cols_v.at[b], out_hbm.at[0, :, pl.ds(b0, _BW)], ssems[b]
        ).wait()

    lane = lax.iota(jnp.int32, _L)
    col_c = [jnp.full((_L,), d, jnp.int32) for d in range(_DIM)]

    def transpose(b):
        def kbody(k):
            row_idx = lane + k * _L
            for d in range(_DIM):
                v = plsc.load_gather(rows_v.at[b], [row_idx, col_c[d]])
                cols_v[b, d, pl.ds(k * _L, _L)] = v

        plsc.parallel_loop(0, _BW // _L, 1, unroll=8)(kbody)

    fire_gather(0, 0)
    fire_gather(1, 1)

    def hbody(i, carry):
        for b in range(2):
            h = i * 2 + b
            wait_gather(b)

            @pl.when(h >= 2)
            def _():
                wait_store(b)

            transpose(b)
            nxt = h + 2

            @pl.when(nxt < _HIST)
            def _():
                fire_gather(nxt, b)

            fire_store(h, b)
        return carry

    lax.fori_loop(0, _HIST // 2, hbody, 0)
    wait_store(0)
    wait_store(1)


def kernel(inputs, table):
    ids_t = inputs.astype(jnp.int32).T
    out = _gather_kernel(ids_t, table)
    return out.transpose(2, 0, 1)


# speedup vs baseline: 1.1119x; 1.1119x over previous
import functools

import jax
import jax.numpy as jnp
from jax import lax
from jax.experimental import pallas as pl
from jax.experimental.pallas import tpu as pltpu
from jax.experimental.pallas import tpu_sc as plsc

_VOCAB = 1000000
_DIM = 32
_BATCH = 16384
_HIST = 50
_NC = 2
_NS = 16
_NW = _NC * _NS
_BW = _BATCH // _NW

_mesh = plsc.VectorSubcoreMesh(core_axis_name="c", subcore_axis_name="s")


@functools.partial(
    pl.kernel,
    out_type=jax.ShapeDtypeStruct((_HIST, _BATCH, _DIM), jnp.float32),
    mesh=_mesh,
    scratch_types=[
        pltpu.VMEM((_HIST, _BW), jnp.int32),
        pltpu.VMEM((2, _BW, _DIM), jnp.float32),
        pltpu.SemaphoreType.DMA,
        pltpu.SemaphoreType.DMA,
        pltpu.SemaphoreType.DMA,
        pltpu.SemaphoreType.DMA,
    ],
    compiler_params=pltpu.CompilerParams(
        use_tc_tiling_on_sc=False, needs_layout_passes=False
    ),
)
def _gather_kernel(ids_hbm, table_hbm, out_hbm, idx_v, rows_v,
                   g0, g1, s0, s1):
    gsems = (g0, g1)
    ssems = (s0, s1)
    wid = lax.axis_index("s") * _NC + lax.axis_index("c")
    b0 = wid * _BW
    pltpu.sync_copy(ids_hbm.at[:, pl.ds(b0, _BW)], idx_v)

    def fire_gather(h, b):
        pltpu.async_copy(table_hbm.at[idx_v.at[h]], rows_v.at[b], gsems[b])

    def wait_gather(b):
        pltpu.make_async_copy(
            table_hbm.at[idx_v.at[0]], rows_v.at[b], gsems[b]
        ).wait()

    def fire_store(h, b):
        pltpu.async_copy(rows_v.at[b], out_hbm.at[h, pl.ds(b0, _BW), :],
                         ssems[b])

    def wait_store(b):
        pltpu.make_async_copy(
            rows_v.at[b], out_hbm.at[0, pl.ds(b0, _BW), :], ssems[b]
        ).wait()

    fire_gather(0, 0)
    fire_gather(1, 1)

    def hbody(i, carry):
        for b in range(2):
            h = i * 2 + b
            wait_gather(b)
            fire_store(h, b)
            nxt = h + 2

            @pl.when(nxt < _HIST)
            def _():
                wait_store(b)
                fire_gather(nxt, b)

        return carry

    lax.fori_loop(0, _HIST // 2, hbody, 0)
    wait_store(0)
    wait_store(1)


def kernel(inputs, table):
    ids_t = inputs.astype(jnp.int32).T
    out = _gather_kernel(ids_t, table)
    return out.transpose(1, 0, 2)


# speedup vs baseline: 1.1122x; 1.0003x over previous
import functools

import jax
import jax.numpy as jnp
from jax import lax
from jax.experimental import pallas as pl
from jax.experimental.pallas import tpu as pltpu
from jax.experimental.pallas import tpu_sc as plsc

_VOCAB = 1000000
_DIM = 32
_BATCH = 16384
_HIST = 50
_NC = 2
_NS = 16
_NW = _NC * _NS
_BW = _BATCH // _NW

_mesh = plsc.VectorSubcoreMesh(core_axis_name="c", subcore_axis_name="s")


@functools.partial(
    pl.kernel,
    out_type=jax.ShapeDtypeStruct((_HIST, _BATCH, _DIM), jnp.float32),
    mesh=_mesh,
    scratch_types=[
        pltpu.VMEM((_HIST, _BW), jnp.int32),
        pltpu.VMEM((2, _BW, _DIM), jnp.float32),
        pltpu.SemaphoreType.DMA,
        pltpu.SemaphoreType.DMA,
        pltpu.SemaphoreType.DMA,
        pltpu.SemaphoreType.DMA,
    ],
    compiler_params=pltpu.CompilerParams(
        use_tc_tiling_on_sc=False, needs_layout_passes=False
    ),
)
def _gather_kernel(ids_hbm, table_hbm, out_hbm, idx_v, rows_v,
                   g0, g1, s0, s1):
    gsems = (g0, g1)
    ssems = (s0, s1)
    wid = lax.axis_index("s") * _NC + lax.axis_index("c")
    b0 = wid * _BW
    pltpu.sync_copy(ids_hbm.at[:, pl.ds(b0, _BW)], idx_v)

    def fire_gather(h, b):
        pltpu.async_copy(table_hbm.at[idx_v.at[h]], rows_v.at[b], gsems[b])

    def wait_gather(b):
        pltpu.make_async_copy(
            table_hbm.at[idx_v.at[0]], rows_v.at[b], gsems[b]
        ).wait()

    def fire_store(h, b):
        pltpu.async_copy(rows_v.at[b], out_hbm.at[h, pl.ds(b0, _BW), :],
                         ssems[b])

    def wait_store(b):
        pltpu.make_async_copy(
            rows_v.at[b], out_hbm.at[0, pl.ds(b0, _BW), :], ssems[b]
        ).wait()

    fire_gather(0, 0)
    fire_gather(1, 1)

    def hbody(i, carry):
        for b in range(2):
            h = i * 2 + b
            wait_gather(b)
            fire_store(h, b)
            nxt = h + 2

            @pl.when(nxt < _HIST)
            def _():
                wait_store(b)
                fire_gather(nxt, b)

        return carry

    lax.fori_loop(0, _HIST // 2, hbody, 0)
    wait_store(0)
    wait_store(1)


def kernel(inputs, table):
    zero = lax.optimization_barrier(jnp.int32(0))
    ids_t = inputs.astype(jnp.int32).T + zero
    out = _gather_kernel(ids_t, table)
    return lax.optimization_barrier(out.transpose(1, 0, 2))
